# baseline (speedup 1.0000x reference)
import functools

import jax
import jax.numpy as jnp
from jax import lax
from jax.experimental import pallas as pl
from jax.experimental.pallas import tpu as pltpu
from jax.experimental.pallas import tpu_sc as plsc

N = 10000
E = 320000
D_EDGE = 16
D_NODE = 128
OUT = 128

NC = 2
NS = 16
NW = NC * NS

SUB = 128
K = 16
CHUNK = K * SUB

NSLICE = 1
E_SL = E // NSLICE

N_PAD = 10112
ZROWS = N_PAD // NS


def _geom(n):
    t_cnt = -(-n // (NW * CHUNK))
    per_tile = t_cnt * CHUNK
    e_pad = NW * per_tile
    b_part = (n // CHUNK) * CHUNK
    clamp = n - CHUNK
    shift = b_part - clamp
    tail = n - b_part
    return t_cnt, per_tile, e_pad, b_part, clamp, shift, tail


def _make_sc_body(n):
    t_cnt, per_tile, _, _, clamp, _, _ = _geom(n)

    def body(edge_hbm, recv_hbm, send_hbm, zero_hbm,
             pin_hbm, pout_hbm,
             rows_a, rows_b, idxr_a, idxr_b, idxs_a, idxs_b,
             agg_in, agg_out,
             sem_a, sem_b, sem_sc):
        c = lax.axis_index("c")
        s = lax.axis_index("s")
        wid = c * NS + s

        @pl.when(s == 0)
        def _():
            pltpu.sync_copy(zero_hbm, agg_in)

        @pl.when(s == 1)
        def _():
            pltpu.sync_copy(zero_hbm, agg_out)

        plsc.subcore_barrier()

        def start_stage(t, rows_v, idxr_v, idxs_v, sem):
            b = wid * per_tile + t * CHUNK
            row_base = jnp.minimum(b, clamp)
            pltpu.async_copy(edge_hbm.at[pl.ds(row_base, CHUNK)], rows_v, sem)
            pltpu.async_copy(recv_hbm.at[pl.ds(b, CHUNK)], idxr_v, sem)
            pltpu.async_copy(send_hbm.at[pl.ds(b, CHUNK)], idxs_v, sem)

        def wait_stage(rows_v, idxr_v, idxs_v, sem):
            pltpu.make_async_copy(
                edge_hbm.at[pl.ds(0, CHUNK)], rows_v, sem).wait()
            pltpu.make_async_copy(
                recv_hbm.at[pl.ds(0, CHUNK)], idxr_v, sem).wait()
            pltpu.make_async_copy(
                send_hbm.at[pl.ds(0, CHUNK)], idxs_v, sem).wait()

        def do_chunk(t, rows_v, idxr_v, idxs_v, sem,
                     rows_n, idxr_n, idxs_n, sem_n):
            wait_stage(rows_v, idxr_v, idxs_v, sem)

            @pl.when(t + 1 < t_cnt)
            def _():
                start_stage(t + 1, rows_n, idxr_n, idxs_n, sem_n)

            def sub_body(j, _):
                src = rows_v.at[pl.ds(j * SUB, SUB)]
                ix = pl.ds(j * SUB, SUB)
                pltpu.async_copy(src, agg_in.at[idxr_v.at[ix]], sem_sc,
                                 add=True)
                pltpu.async_copy(src, agg_out.at[idxs_v.at[ix]], sem_sc,
                                 add=True)
                return 0

            lax.fori_loop(0, K, sub_body, 0)
            pltpu.make_async_copy(
                edge_hbm.at[pl.ds(0, CHUNK)], rows_v, sem_sc).wait()
            pltpu.make_async_copy(
                edge_hbm.at[pl.ds(0, CHUNK)], rows_v, sem_sc).wait()

        start_stage(0, rows_a, idxr_a, idxs_a, sem_a)

        def outer(t, _):
            @pl.when(t % 2 == 0)
            def _():
                do_chunk(t, rows_a, idxr_a, idxs_a, sem_a,
                         rows_b, idxr_b, idxs_b, sem_b)

            @pl.when(t % 2 == 1)
            def _():
                do_chunk(t, rows_b, idxr_b, idxs_b, sem_b,
                         rows_a, idxr_a, idxs_a, sem_a)

            return 0

        lax.fori_loop(0, t_cnt, outer, 0)
        plsc.subcore_barrier()

        @pl.when(s < NS - 1)
        def _():
            sl = pl.ds(s * ZROWS, ZROWS)
            pltpu.sync_copy(agg_in.at[sl], pin_hbm.at[c].at[sl])
            pltpu.sync_copy(agg_out.at[sl], pout_hbm.at[c].at[sl])

        @pl.when(s == NS - 1)
        def _():
            tail_rows = N - (NS - 1) * ZROWS
            sl = pl.ds((NS - 1) * ZROWS, tail_rows)
            pltpu.sync_copy(agg_in.at[sl], pin_hbm.at[c].at[sl])
            pltpu.sync_copy(agg_out.at[sl], pout_hbm.at[c].at[sl])

    return body


@functools.cache
def _sc_scatter(n):
  return pl.kernel(
    _make_sc_body(n),
    out_type=(
        jax.ShapeDtypeStruct((NC, N, D_EDGE), jnp.float32),
        jax.ShapeDtypeStruct((NC, N, D_EDGE), jnp.float32),
    ),
    mesh=plsc.VectorSubcoreMesh(core_axis_name="c", subcore_axis_name="s",
                                num_cores=NC, num_subcores=NS),
    compiler_params=pltpu.CompilerParams(use_tc_tiling_on_sc=False),
    scratch_types=[
        pltpu.VMEM((CHUNK, D_EDGE), jnp.float32),
        pltpu.VMEM((CHUNK, D_EDGE), jnp.float32),
        pltpu.VMEM((CHUNK,), jnp.int32),
        pltpu.VMEM((CHUNK,), jnp.int32),
        pltpu.VMEM((CHUNK,), jnp.int32),
        pltpu.VMEM((CHUNK,), jnp.int32),
        pltpu.VMEM_SHARED((N_PAD, D_EDGE), jnp.float32),
        pltpu.VMEM_SHARED((N_PAD, D_EDGE), jnp.float32),
        pltpu.SemaphoreType.DMA,
        pltpu.SemaphoreType.DMA,
        pltpu.SemaphoreType.DMA,
    ],
  )


def _pad_idx(ix):
    n = ix.shape[0]
    _, _, e_pad, b_part, _, shift, tail = _geom(n)

    def dummies(count):
        return N + jnp.arange(count, dtype=jnp.int32) % (N_PAD - N)

    return jnp.concatenate([
        ix[:b_part],
        dummies(shift),
        ix[b_part:],
        dummies(e_pad - b_part - shift - tail),
    ])


_BLKE = 12800


def _tc_pack_body(xt, eye, out):
    x = xt[...]
    xT = lax.dot_general(x, eye[...], (((0,), (0,)), ((), ())),
                         preferred_element_type=jnp.float32,
                         precision="highest")
    xt3 = xT.reshape(_BLKE // 8, 8, D_EDGE)
    out[...] = jnp.concatenate([xt3[:, j, :] for j in range(8)], axis=1)


def _tc_pack(xt):
    return pl.pallas_call(
        _tc_pack_body,
        grid=(E // _BLKE,),
        in_specs=[
            pl.BlockSpec((D_EDGE, _BLKE), lambda i: (0, i)),
            pl.BlockSpec((D_EDGE, D_EDGE), lambda i: (0, 0)),
        ],
        out_specs=pl.BlockSpec((_BLKE // 8, 128), lambda i: (i, 0)),
        out_shape=jax.ShapeDtypeStruct((E // 8, 128), jnp.float32),
    )(xt, jnp.eye(D_EDGE, dtype=jnp.float32))


_GBLK = N // 8
_PP = N * D_EDGE // 128


def _tc_linear_body(nf3, pin1, pout1, wnt, wci, wco, b, out3):
    pi = pin1[0] + pin1[1]
    po = pout1[0] + pout1[1]
    acc = lax.dot_general(nf3[...], wnt[...],
                          dimension_numbers=(((2,), (0,)), ((), ())),
                          preferred_element_type=jnp.float32,
                          precision="highest")
    acc = acc + lax.dot_general(pi, wci[...],
                                dimension_numbers=(((1,), (0,)), ((), ())),
                                preferred_element_type=jnp.float32,
                                precision="highest")
    acc = acc + lax.dot_general(po, wco[...],
                                dimension_numbers=(((1,), (0,)), ((), ())),
                                preferred_element_type=jnp.float32,
                                precision="highest")
    out3[...] = acc + b[...]


def _tc_linear(nf3, pin1, pout1, wnt, wci, wco, bias3d):
    pspec = pl.BlockSpec((NC, _PP, 128), lambda i: (0, 0, 0))
    return pl.pallas_call(
        _tc_linear_body,
        grid=(1,),
        in_specs=[
            pl.BlockSpec((_GBLK, 8, D_NODE), lambda i: (0, 0, 0)),
            pspec, pspec,
            pl.BlockSpec((D_NODE, OUT), lambda i: (0, 0)),
            pl.BlockSpec((128, 8, OUT), lambda i: (0, 0, 0)),
            pl.BlockSpec((128, 8, OUT), lambda i: (0, 0, 0)),
            pl.BlockSpec((1, 1, OUT), lambda i: (0, 0, 0)),
        ],
        out_specs=pl.BlockSpec((_GBLK, 8, OUT), lambda i: (0, 0, 0)),
        out_shape=jax.ShapeDtypeStruct((N // 8, 8, OUT), jnp.float32),
    )(nf3, pin1, pout1, wnt, wci, wco, bias3d)


def _lift_w(w):
    eye = jnp.eye(8, dtype=jnp.float32)
    wc = eye[:, None, :, None] * w.T[None, :, None, :]
    return wc.reshape(128, 8, OUT)


def kernel(node_features, edge_features, senders, receivers,
           W_node, W_incoming, W_outgoing, bias):
    zeros = jnp.zeros((N_PAD, D_EDGE), jnp.float32)
    edge_rows = _tc_pack(edge_features.T).reshape(E, D_EDGE)
    pin, pout = _sc_scatter(E_SL)(
        edge_rows, _pad_idx(receivers), _pad_idx(senders), zeros)
    out3 = _tc_linear(node_features.reshape(N // 8, 8, D_NODE),
                      pin.reshape(NC, _PP, 128), pout.reshape(NC, _PP, 128),
                      W_node.T, _lift_w(W_incoming), _lift_w(W_outgoing),
                      bias.reshape(1, 1, OUT))
    return out3.reshape(N, OUT)


# speedup vs baseline: 1.3566x; 1.3566x over previous
import functools

import jax
import jax.numpy as jnp
from jax import lax
from jax.experimental import pallas as pl
from jax.experimental.pallas import tpu as pltpu
from jax.experimental.pallas import tpu_sc as plsc

N = 10000
E = 320000
D_EDGE = 16
D_NODE = 128
OUT = 128

NC = 2
NS = 16
NW = NC * NS

SUB = 128
K = 16
CHUNK = K * SUB

NSLICE = 1
E_SL = E // NSLICE

N_PAD = 10112
ZROWS = N_PAD // NS


def _geom(n):
    t_cnt = -(-n // (NW * CHUNK))
    per_tile = t_cnt * CHUNK
    e_pad = NW * per_tile
    b_part = (n // CHUNK) * CHUNK
    clamp = n - CHUNK
    shift = b_part - clamp
    tail = n - b_part
    return t_cnt, per_tile, e_pad, b_part, clamp, shift, tail


def _make_sc_body(n):
    t_cnt, per_tile, _, _, clamp, _, _ = _geom(n)

    def body(edge_hbm, recv_hbm, send_hbm, zero_hbm,
             pin_hbm, pout_hbm,
             rows_a, rows_b, idxr_a, idxr_b, idxs_a, idxs_b,
             agg_in, agg_out,
             sem_a, sem_b, sem_sc):
        c = lax.axis_index("c")
        s = lax.axis_index("s")
        wid = c * NS + s

        @pl.when(s == 0)
        def _():
            pltpu.sync_copy(zero_hbm, agg_in)

        @pl.when(s == 1)
        def _():
            pltpu.sync_copy(zero_hbm, agg_out)

        plsc.subcore_barrier()

        def start_stage(t, rows_v, idxr_v, idxs_v, sem):
            b = wid * per_tile + t * CHUNK
            row_base = jnp.minimum(b, clamp)
            pltpu.async_copy(edge_hbm.at[pl.ds(row_base, CHUNK)], rows_v, sem)
            pltpu.async_copy(recv_hbm.at[pl.ds(b, CHUNK)], idxr_v, sem)
            pltpu.async_copy(send_hbm.at[pl.ds(b, CHUNK)], idxs_v, sem)

        def wait_stage(rows_v, idxr_v, idxs_v, sem):
            pltpu.make_async_copy(
                edge_hbm.at[pl.ds(0, CHUNK)], rows_v, sem).wait()
            pltpu.make_async_copy(
                recv_hbm.at[pl.ds(0, CHUNK)], idxr_v, sem).wait()
            pltpu.make_async_copy(
                send_hbm.at[pl.ds(0, CHUNK)], idxs_v, sem).wait()

        def do_chunk(t, rows_v, idxr_v, idxs_v, sem,
                     rows_n, idxr_n, idxs_n, sem_n):
            wait_stage(rows_v, idxr_v, idxs_v, sem)

            @pl.when(t + 1 < t_cnt)
            def _():
                start_stage(t + 1, rows_n, idxr_n, idxs_n, sem_n)

            def sub_body(j, _):
                src = rows_v.at[pl.ds(j * SUB, SUB)]
                ix = pl.ds(j * SUB, SUB)
                pltpu.async_copy(src, agg_in.at[idxr_v.at[ix]], sem_sc,
                                 add=True)
                pltpu.async_copy(src, agg_out.at[idxs_v.at[ix]], sem_sc,
                                 add=True)
                return 0

            lax.fori_loop(0, K, sub_body, 0)
            pltpu.make_async_copy(
                edge_hbm.at[pl.ds(0, CHUNK)], rows_v, sem_sc).wait()
            pltpu.make_async_copy(
                edge_hbm.at[pl.ds(0, CHUNK)], rows_v, sem_sc).wait()

        start_stage(0, rows_a, idxr_a, idxs_a, sem_a)

        def outer(t, _):
            @pl.when(t % 2 == 0)
            def _():
                do_chunk(t, rows_a, idxr_a, idxs_a, sem_a,
                         rows_b, idxr_b, idxs_b, sem_b)

            @pl.when(t % 2 == 1)
            def _():
                do_chunk(t, rows_b, idxr_b, idxs_b, sem_b,
                         rows_a, idxr_a, idxs_a, sem_a)

            return 0

        lax.fori_loop(0, t_cnt, outer, 0)
        plsc.subcore_barrier()

        @pl.when(s < NS - 1)
        def _():
            sl = pl.ds(s * ZROWS, ZROWS)
            pltpu.sync_copy(agg_in.at[sl], pin_hbm.at[c].at[sl])
            pltpu.sync_copy(agg_out.at[sl], pout_hbm.at[c].at[sl])

        @pl.when(s == NS - 1)
        def _():
            tail_rows = N - (NS - 1) * ZROWS
            sl = pl.ds((NS - 1) * ZROWS, tail_rows)
            pltpu.sync_copy(agg_in.at[sl], pin_hbm.at[c].at[sl])
            pltpu.sync_copy(agg_out.at[sl], pout_hbm.at[c].at[sl])

    return body


@functools.cache
def _sc_scatter(n):
  return pl.kernel(
    _make_sc_body(n),
    out_type=(
        jax.ShapeDtypeStruct((NC, N, D_EDGE), jnp.float32),
        jax.ShapeDtypeStruct((NC, N, D_EDGE), jnp.float32),
    ),
    mesh=plsc.VectorSubcoreMesh(core_axis_name="c", subcore_axis_name="s",
                                num_cores=NC, num_subcores=NS),
    compiler_params=pltpu.CompilerParams(use_tc_tiling_on_sc=False),
    scratch_types=[
        pltpu.VMEM((CHUNK, D_EDGE), jnp.float32),
        pltpu.VMEM((CHUNK, D_EDGE), jnp.float32),
        pltpu.VMEM((CHUNK,), jnp.int32),
        pltpu.VMEM((CHUNK,), jnp.int32),
        pltpu.VMEM((CHUNK,), jnp.int32),
        pltpu.VMEM((CHUNK,), jnp.int32),
        pltpu.VMEM_SHARED((N_PAD, D_EDGE), jnp.float32),
        pltpu.VMEM_SHARED((N_PAD, D_EDGE), jnp.float32),
        pltpu.SemaphoreType.DMA,
        pltpu.SemaphoreType.DMA,
        pltpu.SemaphoreType.DMA,
    ],
  )


def _pad_idx(ix):
    n = ix.shape[0]
    _, _, e_pad, b_part, _, shift, tail = _geom(n)

    def dummies(count):
        return N + jnp.arange(count, dtype=jnp.int32) % (N_PAD - N)

    return jnp.concatenate([
        ix[:b_part],
        dummies(shift),
        ix[b_part:],
        dummies(e_pad - b_part - shift - tail),
    ])


_GBLK = N // 8
_PP = N * D_EDGE // 128


def _tc_linear_body(nf3, pin1, pout1, wnt, wci, wco, b, out3):
    pi = pin1[0] + pin1[1]
    po = pout1[0] + pout1[1]
    acc = lax.dot_general(nf3[...], wnt[...],
                          dimension_numbers=(((2,), (0,)), ((), ())),
                          preferred_element_type=jnp.float32,
                          precision="highest")
    acc = acc + lax.dot_general(pi, wci[...],
                                dimension_numbers=(((1,), (0,)), ((), ())),
                                preferred_element_type=jnp.float32,
                                precision="highest")
    acc = acc + lax.dot_general(po, wco[...],
                                dimension_numbers=(((1,), (0,)), ((), ())),
                                preferred_element_type=jnp.float32,
                                precision="highest")
    out3[...] = acc + b[...]


def _tc_linear(nf3, pin1, pout1, wnt, wci, wco, bias3d):
    pspec = pl.BlockSpec((NC, _PP, 128), lambda i: (0, 0, 0))
    return pl.pallas_call(
        _tc_linear_body,
        grid=(1,),
        in_specs=[
            pl.BlockSpec((_GBLK, 8, D_NODE), lambda i: (0, 0, 0)),
            pspec, pspec,
            pl.BlockSpec((D_NODE, OUT), lambda i: (0, 0)),
            pl.BlockSpec((128, 8, OUT), lambda i: (0, 0, 0)),
            pl.BlockSpec((128, 8, OUT), lambda i: (0, 0, 0)),
            pl.BlockSpec((1, 1, OUT), lambda i: (0, 0, 0)),
        ],
        out_specs=pl.BlockSpec((_GBLK, 8, OUT), lambda i: (0, 0, 0)),
        out_shape=jax.ShapeDtypeStruct((N // 8, 8, OUT), jnp.float32),
    )(nf3, pin1, pout1, wnt, wci, wco, bias3d)


def _lift_w(w):
    eye = jnp.eye(8, dtype=jnp.float32)
    wc = eye[:, None, :, None] * w.T[None, :, None, :]
    return wc.reshape(128, 8, OUT)


def kernel(node_features, edge_features, senders, receivers,
           W_node, W_incoming, W_outgoing, bias):
    zeros = jnp.zeros((N_PAD, D_EDGE), jnp.float32)
    pin, pout = _sc_scatter(E_SL)(
        edge_features, _pad_idx(receivers), _pad_idx(senders), zeros)
    out3 = _tc_linear(node_features.reshape(N // 8, 8, D_NODE),
                      pin.reshape(NC, _PP, 128), pout.reshape(NC, _PP, 128),
                      W_node.T, _lift_w(W_incoming), _lift_w(W_outgoing),
                      bias.reshape(1, 1, OUT))
    return out3.reshape(N, OUT)


# speedup vs baseline: 1.3865x; 1.0221x over previous
import functools

import jax
import jax.numpy as jnp
from jax import lax
from jax.experimental import pallas as pl
from jax.experimental.pallas import tpu as pltpu
from jax.experimental.pallas import tpu_sc as plsc

N = 10000
E = 320000
D_EDGE = 16
D_NODE = 128
OUT = 128

NC = 2
NS = 16
NW = NC * NS

SUB = 128
K = 16
CHUNK = K * SUB

NSLICE = 1
E_SL = E // NSLICE

N_PAD = 10112
ZROWS = N_PAD // NS


def _geom(n):
    t_cnt = -(-n // (NW * CHUNK))
    per_tile = t_cnt * CHUNK
    e_pad = NW * per_tile
    b_part = (n // CHUNK) * CHUNK
    clamp = n - CHUNK
    shift = b_part - clamp
    tail = n - b_part
    return t_cnt, per_tile, e_pad, b_part, clamp, shift, tail


def _make_sc_body(n):
    t_cnt, per_tile, _, _, clamp, _, _ = _geom(n)

    def body(edge_hbm, recv_hbm, send_hbm, zero_hbm,
             pin_hbm, pout_hbm,
             rows_a, rows_b, idxr_a, idxr_b, idxs_a, idxs_b,
             agg_in, agg_out,
             sem_a, sem_b, sem_sc):
        c = lax.axis_index("c")
        s = lax.axis_index("s")
        wid = c * NS + s

        @pl.when(s == 0)
        def _():
            pltpu.sync_copy(zero_hbm, agg_in)

        @pl.when(s == 1)
        def _():
            pltpu.sync_copy(zero_hbm, agg_out)

        plsc.subcore_barrier()

        def start_stage(t, rows_v, idxr_v, idxs_v, sem):
            b = wid * per_tile + t * CHUNK
            row_base = jnp.minimum(b, clamp)
            pltpu.async_copy(edge_hbm.at[pl.ds(row_base, CHUNK)], rows_v, sem)
            pltpu.async_copy(recv_hbm.at[pl.ds(b, CHUNK)], idxr_v, sem)
            pltpu.async_copy(send_hbm.at[pl.ds(b, CHUNK)], idxs_v, sem)

        def wait_stage(rows_v, idxr_v, idxs_v, sem):
            pltpu.make_async_copy(
                edge_hbm.at[pl.ds(0, CHUNK)], rows_v, sem).wait()
            pltpu.make_async_copy(
                recv_hbm.at[pl.ds(0, CHUNK)], idxr_v, sem).wait()
            pltpu.make_async_copy(
                send_hbm.at[pl.ds(0, CHUNK)], idxs_v, sem).wait()

        def do_chunk(t, rows_v, idxr_v, idxs_v, sem,
                     rows_n, idxr_n, idxs_n, sem_n):
            wait_stage(rows_v, idxr_v, idxs_v, sem)

            @pl.when(t + 1 < t_cnt)
            def _():
                start_stage(t + 1, rows_n, idxr_n, idxs_n, sem_n)

            def sub_body(j, _):
                src = rows_v.at[pl.ds(j * SUB, SUB)]
                ix = pl.ds(j * SUB, SUB)
                pltpu.async_copy(src, agg_in.at[idxr_v.at[ix]], sem_sc,
                                 add=True)
                pltpu.async_copy(src, agg_out.at[idxs_v.at[ix]], sem_sc,
                                 add=True)
                return 0

            lax.fori_loop(0, K, sub_body, 0)
            pltpu.make_async_copy(
                edge_hbm.at[pl.ds(0, CHUNK)], rows_v, sem_sc).wait()
            pltpu.make_async_copy(
                edge_hbm.at[pl.ds(0, CHUNK)], rows_v, sem_sc).wait()

        start_stage(0, rows_a, idxr_a, idxs_a, sem_a)

        def outer(t, _):
            @pl.when(t % 2 == 0)
            def _():
                do_chunk(t, rows_a, idxr_a, idxs_a, sem_a,
                         rows_b, idxr_b, idxs_b, sem_b)

            @pl.when(t % 2 == 1)
            def _():
                do_chunk(t, rows_b, idxr_b, idxs_b, sem_b,
                         rows_a, idxr_a, idxs_a, sem_a)

            return 0

        lax.fori_loop(0, t_cnt, outer, 0)
        plsc.subcore_barrier()

        @pl.when(s < NS - 1)
        def _():
            sl = pl.ds(s * ZROWS, ZROWS)
            pltpu.sync_copy(agg_in.at[sl], pin_hbm.at[c].at[sl])
            pltpu.sync_copy(agg_out.at[sl], pout_hbm.at[c].at[sl])

        @pl.when(s == NS - 1)
        def _():
            tail_rows = N - (NS - 1) * ZROWS
            sl = pl.ds((NS - 1) * ZROWS, tail_rows)
            pltpu.sync_copy(agg_in.at[sl], pin_hbm.at[c].at[sl])
            pltpu.sync_copy(agg_out.at[sl], pout_hbm.at[c].at[sl])

    return body


@functools.cache
def _sc_scatter(n):
  return pl.kernel(
    _make_sc_body(n),
    out_type=(
        jax.ShapeDtypeStruct((NC, N, D_EDGE), jnp.float32),
        jax.ShapeDtypeStruct((NC, N, D_EDGE), jnp.float32),
    ),
    mesh=plsc.VectorSubcoreMesh(core_axis_name="c", subcore_axis_name="s",
                                num_cores=NC, num_subcores=NS),
    compiler_params=pltpu.CompilerParams(use_tc_tiling_on_sc=False),
    scratch_types=[
        pltpu.VMEM((CHUNK, D_EDGE), jnp.float32),
        pltpu.VMEM((CHUNK, D_EDGE), jnp.float32),
        pltpu.VMEM((CHUNK,), jnp.int32),
        pltpu.VMEM((CHUNK,), jnp.int32),
        pltpu.VMEM((CHUNK,), jnp.int32),
        pltpu.VMEM((CHUNK,), jnp.int32),
        pltpu.VMEM_SHARED((N_PAD, D_EDGE), jnp.float32),
        pltpu.VMEM_SHARED((N_PAD, D_EDGE), jnp.float32),
        pltpu.SemaphoreType.DMA,
        pltpu.SemaphoreType.DMA,
        pltpu.SemaphoreType.DMA,
    ],
  )


def _pad_idx(ix):
    n = ix.shape[0]
    _, _, e_pad, b_part, _, shift, tail = _geom(n)

    def dummies(count):
        return N + jnp.arange(count, dtype=jnp.int32) % (N_PAD - N)

    return jnp.concatenate([
        ix[:b_part],
        dummies(shift),
        ix[b_part:],
        dummies(e_pad - b_part - shift - tail),
    ])


_GBLK = N // 8
_PP = N * D_EDGE // 128


_NBLK = 250


def _tc_node_body(nf3, wnt, b, out3):
    acc = lax.dot_general(nf3[...], wnt[...],
                          dimension_numbers=(((2,), (0,)), ((), ())),
                          preferred_element_type=jnp.float32,
                          precision="highest")
    out3[...] = acc + b[...]


def _tc_node(nf3, wnt, bias3d):
    return pl.pallas_call(
        _tc_node_body,
        grid=(N // 8 // _NBLK,),
        in_specs=[
            pl.BlockSpec((_NBLK, 8, D_NODE), lambda i: (i, 0, 0)),
            pl.BlockSpec((D_NODE, OUT), lambda i: (0, 0)),
            pl.BlockSpec((1, 1, OUT), lambda i: (0, 0, 0)),
        ],
        out_specs=pl.BlockSpec((_NBLK, 8, OUT), lambda i: (i, 0, 0)),
        out_shape=jax.ShapeDtypeStruct((N // 8, 8, OUT), jnp.float32),
    )(nf3, wnt, bias3d)


def _tc_agg_body(base3, pin1, pout1, wci, wco, out3):
    pi = pin1[0] + pin1[1]
    po = pout1[0] + pout1[1]
    acc = base3[...] + lax.dot_general(
        pi, wci[...], dimension_numbers=(((1,), (0,)), ((), ())),
        preferred_element_type=jnp.float32, precision="highest")
    acc = acc + lax.dot_general(
        po, wco[...], dimension_numbers=(((1,), (0,)), ((), ())),
        preferred_element_type=jnp.float32, precision="highest")
    out3[...] = acc


def _tc_agg(base3, pin1, pout1, wci, wco):
    pspec = pl.BlockSpec((NC, _PP, 128), lambda i: (0, 0, 0))
    return pl.pallas_call(
        _tc_agg_body,
        grid=(1,),
        in_specs=[
            pl.BlockSpec((_GBLK, 8, OUT), lambda i: (0, 0, 0)),
            pspec, pspec,
            pl.BlockSpec((128, 8, OUT), lambda i: (0, 0, 0)),
            pl.BlockSpec((128, 8, OUT), lambda i: (0, 0, 0)),
        ],
        out_specs=pl.BlockSpec((_GBLK, 8, OUT), lambda i: (0, 0, 0)),
        out_shape=jax.ShapeDtypeStruct((N // 8, 8, OUT), jnp.float32),
    )(base3, pin1, pout1, wci, wco)


def _lift_w(w):
    eye = jnp.eye(8, dtype=jnp.float32)
    wc = eye[:, None, :, None] * w.T[None, :, None, :]
    return wc.reshape(128, 8, OUT)


def kernel(node_features, edge_features, senders, receivers,
           W_node, W_incoming, W_outgoing, bias):
    zeros = jnp.zeros((N_PAD, D_EDGE), jnp.float32)
    base3 = _tc_node(node_features.reshape(N // 8, 8, D_NODE),
                     W_node.T, bias.reshape(1, 1, OUT))
    pin, pout = _sc_scatter(E_SL)(
        edge_features, _pad_idx(receivers), _pad_idx(senders), zeros)
    out3 = _tc_agg(base3,
                   pin.reshape(NC, _PP, 128), pout.reshape(NC, _PP, 128),
                   _lift_w(W_incoming), _lift_w(W_outgoing))
    return out3.reshape(N, OUT)
